# Initial kernel scaffold; baseline (speedup 1.0000x reference)
import jax
import jax.numpy as jnp
from jax.experimental import pallas as pl


def kernel(features, edge_index, W1, b1, W2, b2, W3, b3):
    raise NotImplementedError("write your pallas kernel here")


# speedup vs baseline: 6.2048x; 6.2048x over previous
import functools

import jax
import jax.numpy as jnp
from jax import lax
from jax.experimental import pallas as pl
from jax.experimental.pallas import tpu as pltpu
from jax.experimental.pallas import tpu_sc as plsc

N = 10000
E = 320000
NP = 10240
NC = 2
NS = 16
NW = NC * NS
EPT = E // NW
CH = 80
TCH = EPT // CH
RPT = NP // NS

_f32 = jnp.float32
_i32 = jnp.int32


def _sc_mesh():
    return plsc.VectorSubcoreMesh(core_axis_name="c", subcore_axis_name="s")


DW = 16


def _make_count():
    @functools.partial(
        pl.kernel,
        out_type=[jax.ShapeDtypeStruct((NC, NP, DW), _f32),
                  jax.ShapeDtypeStruct((NC, NP, DW), _f32)],
        mesh=_sc_mesh(),
        scratch_types=[
            pltpu.VMEM((TCH, CH), _i32),
            pltpu.VMEM((TCH, CH), _i32),
            pltpu.VMEM((CH, DW), _f32),
            pltpu.VMEM((RPT, DW), _f32),
            pltpu.VMEM_SHARED((NP, DW), _f32),
            pltpu.VMEM_SHARED((NP, DW), _f32),
        ],
        compiler_params=pltpu.CompilerParams(use_tc_tiling_on_sc=False),
    )
    def cnt(srcs, dsts, ones_h, zeros_h, out_s, out_d,
            src_v, dst_v, ones_v, zv, acc_s, acc_d):
        c = lax.axis_index("c")
        s = lax.axis_index("s")
        w = c * NS + s
        pltpu.sync_copy(srcs.at[w], src_v)
        pltpu.sync_copy(dsts.at[w], dst_v)
        pltpu.sync_copy(ones_h, ones_v)
        pltpu.sync_copy(zeros_h, zv)
        pltpu.sync_copy(zv, acc_s.at[pl.ds(s * RPT, RPT)])
        pltpu.sync_copy(zv, acc_d.at[pl.ds(s * RPT, RPT)])
        plsc.subcore_barrier()

        def body(j, carry):
            pltpu.sync_copy(ones_v, acc_s.at[src_v.at[j]], add=True)
            pltpu.sync_copy(ones_v, acc_d.at[dst_v.at[j]], add=True)
            return carry

        lax.fori_loop(0, TCH, body, 0)
        plsc.subcore_barrier()
        pltpu.sync_copy(acc_s.at[pl.ds(s * RPT, RPT)],
                        out_s.at[c, pl.ds(s * RPT, RPT)])
        pltpu.sync_copy(acc_d.at[pl.ds(s * RPT, RPT)],
                        out_d.at[c, pl.ds(s * RPT, RPT)])

    return cnt


def _make_propagate(width):
    @functools.partial(
        pl.kernel,
        out_type=jax.ShapeDtypeStruct((NC, NP, width), _f32),
        mesh=_sc_mesh(),
        scratch_types=[
            pltpu.VMEM((CH, width), _f32),
            pltpu.VMEM((TCH, CH), _i32),
            pltpu.VMEM((TCH, CH), _i32),
            pltpu.VMEM((128, width), _f32),
            pltpu.VMEM_SHARED((NP, width), _f32),
            pltpu.SemaphoreType.DMA,
        ],
        compiler_params=pltpu.CompilerParams(use_tc_tiling_on_sc=False),
    )
    def prop(table, srcs, dsts, zrows, out, rows_v, src_v, dst_v, zv, acc,
             sem):
        c = lax.axis_index("c")
        s = lax.axis_index("s")
        w = c * NS + s
        pltpu.sync_copy(srcs.at[w], src_v)
        pltpu.sync_copy(dsts.at[w], dst_v)
        pltpu.sync_copy(zrows, zv)
        for t in range(RPT // 128):
            pltpu.sync_copy(zv, acc.at[pl.ds(s * RPT + t * 128, 128)])
        plsc.subcore_barrier()

        def body(j, carry):
            pltpu.async_copy(table.at[src_v.at[j]], rows_v, sem).wait()
            pltpu.sync_copy(rows_v, acc.at[dst_v.at[j]], add=True)
            return carry

        lax.fori_loop(0, TCH, body, 0)
        plsc.subcore_barrier()
        pltpu.sync_copy(acc.at[pl.ds(s * RPT, RPT)],
                        out.at[c, pl.ds(s * RPT, RPT)])

    return prop


_R = 512
_G = NP // _R


def _ln(r):
    mu = jnp.mean(r, axis=1, keepdims=True)
    var = jnp.mean((r - mu) ** 2, axis=1, keepdims=True)
    return (r - mu) * lax.rsqrt(var + 1e-5)


def _row_spec(width):
    return pl.BlockSpec((_R, width), lambda i: (i, 0))


def _full_spec(shape):
    return pl.BlockSpec(shape, lambda i: tuple(0 for _ in shape))


def _tc_scale(x, scale_b):
    def body(x_ref, s_ref, o_ref):
        o_ref[...] = x_ref[...] * s_ref[...]

    return pl.pallas_call(
        body,
        grid=(_G,),
        in_specs=[_row_spec(128), _row_spec(128)],
        out_specs=_row_spec(128),
        out_shape=jax.ShapeDtypeStruct((NP, 128), _f32),
    )(x, scale_b)


def _tc_layer1(pa0, pa1, dnin_b, dnout_b, W1, b1):
    def body(a0, a1, di, do, w_ref, b_ref, oa, ob):
        agg = (a0[...] + a1[...]) * di[...]
        t = jnp.dot(agg, w_ref[...], preferred_element_type=_f32)
        t = t + b_ref[0:1, :]
        h = _ln(jnp.maximum(t, 0.0))
        dob = do[...]
        oa[...] = h[:, :128] * dob
        ob[...] = h[:, 128:] * dob

    return pl.pallas_call(
        body,
        grid=(_G,),
        in_specs=[_row_spec(128), _row_spec(128), _row_spec(128),
                  _row_spec(128), _full_spec((128, 256)), _full_spec((8, 256))],
        out_specs=[_row_spec(128), _row_spec(128)],
        out_shape=[jax.ShapeDtypeStruct((NP, 128), _f32),
                   jax.ShapeDtypeStruct((NP, 128), _f32)],
    )(pa0, pa1, dnin_b, dnout_b, W1, b1)


def _tc_layer2(pa0, pa1, pb0, pb1, dnin_b, dnout_b, W2, b2, W3):
    def body(a0, a1, b0, b1_, di, do, w2_ref, bias_ref, w3_ref, og):
        dib = di[...]
        agg_lo = (a0[...] + a1[...]) * dib
        agg_hi = (b0[...] + b1_[...]) * dib
        t = (jnp.dot(agg_lo, w2_ref[:128, :], preferred_element_type=_f32)
             + jnp.dot(agg_hi, w2_ref[128:, :], preferred_element_type=_f32))
        t = t + bias_ref[0:1, :]
        h = _ln(jnp.maximum(t, 0.0))
        dob = do[...]
        hs_lo = h[:, :128] * dob
        hs_hi = h[:, 128:] * dob
        og[...] = (jnp.dot(hs_lo, w3_ref[:128, :], preferred_element_type=_f32)
                   + jnp.dot(hs_hi, w3_ref[128:, :],
                             preferred_element_type=_f32))

    return pl.pallas_call(
        body,
        grid=(_G,),
        in_specs=[_row_spec(128), _row_spec(128), _row_spec(128),
                  _row_spec(128), _row_spec(128), _row_spec(128),
                  _full_spec((256, 256)), _full_spec((8, 256)),
                  _full_spec((256, 64))],
        out_specs=_row_spec(64),
        out_shape=jax.ShapeDtypeStruct((NP, 64), _f32),
    )(pa0, pa1, pb0, pb1, dnin_b, dnout_b, W2, b2, W3)


def _tc_layer3(q0, q1, dnin_b64, b3):
    def body(a0, a1, di, bias_ref, oy):
        agg = (a0[...] + a1[...]) * di[...]
        t = agg + bias_ref[0:1, :]
        oy[...] = _ln(jnp.maximum(t, 0.0))

    return pl.pallas_call(
        body,
        grid=(_G,),
        in_specs=[_row_spec(64), _row_spec(64), _row_spec(64),
                  _full_spec((8, 64))],
        out_specs=_row_spec(64),
        out_shape=jax.ShapeDtypeStruct((NP, 64), _f32),
    )(q0, q1, dnin_b64, b3)


def kernel(features, edge_index, W1, b1, W2, b2, W3, b3):
    src = edge_index[0].astype(_i32).reshape(NW, TCH, CH)
    dst = edge_index[1].astype(_i32).reshape(NW, TCH, CH)

    feats_p = jnp.pad(features, ((0, NP - N), (0, 0)))
    ones_h = jnp.ones((CH, DW), _f32)
    zeros_dw = jnp.zeros((RPT, DW), _f32)
    zeros_128 = jnp.zeros((128, 128), _f32)
    zeros_64 = jnp.zeros((128, 64), _f32)

    cs, cd = _make_count()(src, dst, ones_h, zeros_dw)
    deg_out = jnp.clip(cs[0, :, 0] + cs[1, :, 0], 1.0, None)
    deg_in = jnp.clip(cd[0, :, 0] + cd[1, :, 0], 1.0, None)
    dn_out = deg_out ** -0.5
    dn_in = deg_in ** -0.5
    dn_out_b = jnp.broadcast_to(dn_out[:, None], (NP, 128))
    dn_in_b = jnp.broadcast_to(dn_in[:, None], (NP, 128))
    dn_in_b64 = jnp.broadcast_to(dn_in[:, None], (NP, 64))

    b1_p = jnp.broadcast_to(b1[None, :], (8, 256))
    b2_p = jnp.broadcast_to(b2[None, :], (8, 256))
    b3_p = jnp.broadcast_to(b3[None, :], (8, 64))

    prop128 = _make_propagate(128)
    prop64 = _make_propagate(64)

    h0s = _tc_scale(feats_p, dn_out_b)
    p1 = prop128(h0s, src, dst, zeros_128)
    h1a, h1b = _tc_layer1(p1[0], p1[1], dn_in_b, dn_out_b, W1, b1_p)

    pa = prop128(h1a, src, dst, zeros_128)
    pb = prop128(h1b, src, dst, zeros_128)
    g = _tc_layer2(pa[0], pa[1], pb[0], pb[1], dn_in_b, dn_out_b, W2, b2_p,
                   W3)

    q = prop64(g, src, dst, zeros_64)
    y = _tc_layer3(q[0], q[1], dn_in_b64, b3_p)
    return y[:N]


# speedup vs baseline: 7.7927x; 1.2559x over previous
import functools

import jax
import jax.numpy as jnp
from jax import lax
from jax.experimental import pallas as pl
from jax.experimental.pallas import tpu as pltpu
from jax.experimental.pallas import tpu_sc as plsc

N = 10000
E = 320000
NP = 10240
NC = 2
NS = 16
NW = NC * NS
EPT = E // NW
CH = 80
TCH = EPT // CH
RPT = NP // NS

_f32 = jnp.float32
_i32 = jnp.int32


def _sc_mesh():
    return plsc.VectorSubcoreMesh(core_axis_name="c", subcore_axis_name="s")


DW = 16


def _make_count():
    @functools.partial(
        pl.kernel,
        out_type=[jax.ShapeDtypeStruct((NC, NP, DW), _f32),
                  jax.ShapeDtypeStruct((NC, NP, DW), _f32)],
        mesh=_sc_mesh(),
        scratch_types=[
            pltpu.VMEM((TCH, CH), _i32),
            pltpu.VMEM((TCH, CH), _i32),
            pltpu.VMEM((CH, DW), _f32),
            pltpu.VMEM((RPT, DW), _f32),
            pltpu.VMEM_SHARED((NP, DW), _f32),
            pltpu.VMEM_SHARED((NP, DW), _f32),
        ],
        compiler_params=pltpu.CompilerParams(use_tc_tiling_on_sc=False),
    )
    def cnt(srcs, dsts, ones_h, zeros_h, out_s, out_d,
            src_v, dst_v, ones_v, zv, acc_s, acc_d):
        c = lax.axis_index("c")
        s = lax.axis_index("s")
        w = c * NS + s
        pltpu.sync_copy(srcs.at[w], src_v)
        pltpu.sync_copy(dsts.at[w], dst_v)
        pltpu.sync_copy(ones_h, ones_v)
        pltpu.sync_copy(zeros_h, zv)
        pltpu.sync_copy(zv, acc_s.at[pl.ds(s * RPT, RPT)])
        pltpu.sync_copy(zv, acc_d.at[pl.ds(s * RPT, RPT)])
        plsc.subcore_barrier()

        def body(j, carry):
            pltpu.sync_copy(ones_v, acc_s.at[src_v.at[j]], add=True)
            pltpu.sync_copy(ones_v, acc_d.at[dst_v.at[j]], add=True)
            return carry

        lax.fori_loop(0, TCH, body, 0)
        plsc.subcore_barrier()
        pltpu.sync_copy(acc_s.at[pl.ds(s * RPT, RPT)],
                        out_s.at[c, pl.ds(s * RPT, RPT)])
        pltpu.sync_copy(acc_d.at[pl.ds(s * RPT, RPT)],
                        out_d.at[c, pl.ds(s * RPT, RPT)])

    return cnt


def _make_propagate(width, depth):
    rem = TCH % depth

    @functools.partial(
        pl.kernel,
        out_type=jax.ShapeDtypeStruct((NC, NP, width), _f32),
        mesh=_sc_mesh(),
        scratch_types=[
            pltpu.VMEM((depth, CH, width), _f32),
            pltpu.VMEM((TCH, CH), _i32),
            pltpu.VMEM((TCH, CH), _i32),
            pltpu.VMEM_SHARED((NP, width), _f32),
        ] + [pltpu.SemaphoreType.DMA] * depth,
        compiler_params=pltpu.CompilerParams(use_tc_tiling_on_sc=False),
    )
    def prop(table, srcs, dsts, zrows, out, rows_v, src_v, dst_v, acc,
             *sems):
        c = lax.axis_index("c")
        s = lax.axis_index("s")
        w = c * NS + s
        pltpu.sync_copy(srcs.at[w], src_v)
        pltpu.sync_copy(dsts.at[w], dst_v)
        pltpu.sync_copy(zrows.at[pl.ds(0, CH)], rows_v.at[0])
        for t in range(RPT // CH):
            pltpu.sync_copy(rows_v.at[0],
                            acc.at[pl.ds(s * RPT + t * CH, CH)])
        plsc.subcore_barrier()

        def body(i, carry):
            j = i * depth
            hs = [pltpu.async_copy(table.at[src_v.at[j + t]], rows_v.at[t],
                                   sems[t])
                  for t in range(depth)]
            for t in range(depth):
                hs[t].wait()
                pltpu.sync_copy(rows_v.at[t], acc.at[dst_v.at[j + t]],
                                add=True)
            return carry

        lax.fori_loop(0, TCH // depth, body, 0)
        for r in range(rem):
            j = TCH - rem + r
            pltpu.async_copy(table.at[src_v.at[j]], rows_v.at[0],
                             sems[0]).wait()
            pltpu.sync_copy(rows_v.at[0], acc.at[dst_v.at[j]], add=True)
        plsc.subcore_barrier()
        pltpu.sync_copy(acc.at[pl.ds(s * RPT, RPT)],
                        out.at[c, pl.ds(s * RPT, RPT)])

    return prop


_R = 512
_G = NP // _R


def _ln(r):
    mu = jnp.mean(r, axis=1, keepdims=True)
    var = jnp.mean((r - mu) ** 2, axis=1, keepdims=True)
    return (r - mu) * lax.rsqrt(var + 1e-5)


def _row_spec(width):
    return pl.BlockSpec((_R, width), lambda i: (i, 0))


def _full_spec(shape):
    return pl.BlockSpec(shape, lambda i: tuple(0 for _ in shape))


def _tc_scale(x, scale_b):
    def body(x_ref, s_ref, o_ref):
        o_ref[...] = x_ref[...] * s_ref[...]

    return pl.pallas_call(
        body,
        grid=(_G,),
        in_specs=[_row_spec(128), _row_spec(128)],
        out_specs=_row_spec(128),
        out_shape=jax.ShapeDtypeStruct((NP, 128), _f32),
    )(x, scale_b)


def _tc_layer1(pa0, pa1, dnin_b, dnout_b, W1, b1):
    def body(a0, a1, di, do, w_ref, b_ref, oa, ob):
        agg = (a0[...] + a1[...]) * di[...]
        t = jnp.dot(agg, w_ref[...], preferred_element_type=_f32)
        t = t + b_ref[0:1, :]
        h = _ln(jnp.maximum(t, 0.0))
        dob = do[...]
        oa[...] = h[:, :128] * dob
        ob[...] = h[:, 128:] * dob

    return pl.pallas_call(
        body,
        grid=(_G,),
        in_specs=[_row_spec(128), _row_spec(128), _row_spec(128),
                  _row_spec(128), _full_spec((128, 256)), _full_spec((8, 256))],
        out_specs=[_row_spec(128), _row_spec(128)],
        out_shape=[jax.ShapeDtypeStruct((NP, 128), _f32),
                   jax.ShapeDtypeStruct((NP, 128), _f32)],
    )(pa0, pa1, dnin_b, dnout_b, W1, b1)


def _tc_layer2(pa0, pa1, pb0, pb1, dnin_b, dnout_b, W2, b2, W3):
    def body(a0, a1, b0, b1_, di, do, w2_ref, bias_ref, w3_ref, og):
        dib = di[...]
        agg_lo = (a0[...] + a1[...]) * dib
        agg_hi = (b0[...] + b1_[...]) * dib
        t = (jnp.dot(agg_lo, w2_ref[:128, :], preferred_element_type=_f32)
             + jnp.dot(agg_hi, w2_ref[128:, :], preferred_element_type=_f32))
        t = t + bias_ref[0:1, :]
        h = _ln(jnp.maximum(t, 0.0))
        dob = do[...]
        hs_lo = h[:, :128] * dob
        hs_hi = h[:, 128:] * dob
        og[...] = (jnp.dot(hs_lo, w3_ref[:128, :], preferred_element_type=_f32)
                   + jnp.dot(hs_hi, w3_ref[128:, :],
                             preferred_element_type=_f32))

    return pl.pallas_call(
        body,
        grid=(_G,),
        in_specs=[_row_spec(128), _row_spec(128), _row_spec(128),
                  _row_spec(128), _row_spec(128), _row_spec(128),
                  _full_spec((256, 256)), _full_spec((8, 256)),
                  _full_spec((256, 64))],
        out_specs=_row_spec(64),
        out_shape=jax.ShapeDtypeStruct((NP, 64), _f32),
    )(pa0, pa1, pb0, pb1, dnin_b, dnout_b, W2, b2, W3)


def _tc_layer3(q0, q1, dnin_b64, b3):
    def body(a0, a1, di, bias_ref, oy):
        agg = (a0[...] + a1[...]) * di[...]
        t = agg + bias_ref[0:1, :]
        oy[...] = _ln(jnp.maximum(t, 0.0))

    return pl.pallas_call(
        body,
        grid=(_G,),
        in_specs=[_row_spec(64), _row_spec(64), _row_spec(64),
                  _full_spec((8, 64))],
        out_specs=_row_spec(64),
        out_shape=jax.ShapeDtypeStruct((NP, 64), _f32),
    )(q0, q1, dnin_b64, b3)


def kernel(features, edge_index, W1, b1, W2, b2, W3, b3):
    src = edge_index[0].astype(_i32).reshape(NW, TCH, CH)
    dst = edge_index[1].astype(_i32).reshape(NW, TCH, CH)

    feats_p = jnp.pad(features, ((0, NP - N), (0, 0)))
    ones_h = jnp.ones((CH, DW), _f32)
    zeros_dw = jnp.zeros((RPT, DW), _f32)
    zeros_128 = jnp.zeros((128, 128), _f32)
    zeros_64 = jnp.zeros((128, 64), _f32)

    cs, cd = _make_count()(src, dst, ones_h, zeros_dw)
    deg_out = jnp.clip(cs[0, :, 0] + cs[1, :, 0], 1.0, None)
    deg_in = jnp.clip(cd[0, :, 0] + cd[1, :, 0], 1.0, None)
    dn_out = deg_out ** -0.5
    dn_in = deg_in ** -0.5
    dn_out_b = jnp.broadcast_to(dn_out[:, None], (NP, 128))
    dn_in_b = jnp.broadcast_to(dn_in[:, None], (NP, 128))
    dn_in_b64 = jnp.broadcast_to(dn_in[:, None], (NP, 64))

    b1_p = jnp.broadcast_to(b1[None, :], (8, 256))
    b2_p = jnp.broadcast_to(b2[None, :], (8, 256))
    b3_p = jnp.broadcast_to(b3[None, :], (8, 64))

    prop128 = _make_propagate(128, 2)
    prop64 = _make_propagate(64, 5)

    h0s = _tc_scale(feats_p, dn_out_b)
    p1 = prop128(h0s, src, dst, zeros_128)
    h1a, h1b = _tc_layer1(p1[0], p1[1], dn_in_b, dn_out_b, W1, b1_p)

    pa = prop128(h1a, src, dst, zeros_128)
    pb = prop128(h1b, src, dst, zeros_128)
    g = _tc_layer2(pa[0], pa[1], pb[0], pb[1], dn_in_b, dn_out_b, W2, b2_p,
                   W3)

    q = prop64(g, src, dst, zeros_64)
    y = _tc_layer3(q[0], q[1], dn_in_b64, b3_p)
    return y[:N]


# speedup vs baseline: 7.9586x; 1.0213x over previous
import functools

import jax
import jax.numpy as jnp
from jax import lax
from jax.experimental import pallas as pl
from jax.experimental.pallas import tpu as pltpu
from jax.experimental.pallas import tpu_sc as plsc

N = 10000
E = 320000
NP = 10240
NC = 2
NS = 16
NW = NC * NS
EPT = E // NW
CH = 80
TCH = EPT // CH
RPT = NP // NS

_f32 = jnp.float32
_i32 = jnp.int32


def _sc_mesh():
    return plsc.VectorSubcoreMesh(core_axis_name="c", subcore_axis_name="s")


DW = 16


def _make_count():
    @functools.partial(
        pl.kernel,
        out_type=[jax.ShapeDtypeStruct((NC, NP, DW), _f32),
                  jax.ShapeDtypeStruct((NC, NP, DW), _f32)],
        mesh=_sc_mesh(),
        scratch_types=[
            pltpu.VMEM((TCH, CH), _i32),
            pltpu.VMEM((TCH, CH), _i32),
            pltpu.VMEM((CH, DW), _f32),
            pltpu.VMEM((RPT, DW), _f32),
            pltpu.VMEM_SHARED((NP, DW), _f32),
            pltpu.VMEM_SHARED((NP, DW), _f32),
        ],
        compiler_params=pltpu.CompilerParams(use_tc_tiling_on_sc=False),
    )
    def cnt(srcs, dsts, ones_h, zeros_h, out_s, out_d,
            src_v, dst_v, ones_v, zv, acc_s, acc_d):
        c = lax.axis_index("c")
        s = lax.axis_index("s")
        w = c * NS + s
        pltpu.sync_copy(srcs.at[w], src_v)
        pltpu.sync_copy(dsts.at[w], dst_v)
        pltpu.sync_copy(ones_h, ones_v)
        pltpu.sync_copy(zeros_h, zv)
        pltpu.sync_copy(zv, acc_s.at[pl.ds(s * RPT, RPT)])
        pltpu.sync_copy(zv, acc_d.at[pl.ds(s * RPT, RPT)])
        plsc.subcore_barrier()

        def body(j, carry):
            pltpu.sync_copy(ones_v, acc_s.at[src_v.at[j]], add=True)
            pltpu.sync_copy(ones_v, acc_d.at[dst_v.at[j]], add=True)
            return carry

        lax.fori_loop(0, TCH, body, 0)
        plsc.subcore_barrier()
        pltpu.sync_copy(acc_s.at[pl.ds(s * RPT, RPT)],
                        out_s.at[c, pl.ds(s * RPT, RPT)])
        pltpu.sync_copy(acc_d.at[pl.ds(s * RPT, RPT)],
                        out_d.at[c, pl.ds(s * RPT, RPT)])

    return cnt


def _make_propagate(width, depth):
    rem = TCH % depth

    @functools.partial(
        pl.kernel,
        out_type=jax.ShapeDtypeStruct((NC, NP, width), _f32),
        mesh=_sc_mesh(),
        scratch_types=[
            pltpu.VMEM((depth, CH, width), _f32),
            pltpu.VMEM((TCH, CH), _i32),
            pltpu.VMEM((TCH, CH), _i32),
            pltpu.VMEM_SHARED((NP, width), _f32),
        ] + [pltpu.SemaphoreType.DMA] * (2 * depth),
        compiler_params=pltpu.CompilerParams(use_tc_tiling_on_sc=False),
    )
    def prop(table, srcs, dsts, zrows, out, rows_v, src_v, dst_v, acc,
             *sems):
        c = lax.axis_index("c")
        s = lax.axis_index("s")
        w = c * NS + s
        pltpu.sync_copy(srcs.at[w], src_v)
        pltpu.sync_copy(dsts.at[w], dst_v)
        pltpu.sync_copy(zrows.at[pl.ds(0, CH)], rows_v.at[0])
        for t in range(RPT // CH):
            pltpu.sync_copy(rows_v.at[0],
                            acc.at[pl.ds(s * RPT + t * CH, CH)])
        plsc.subcore_barrier()

        def body(i, carry):
            j = i * depth
            gh = [pltpu.async_copy(table.at[src_v.at[j + t]], rows_v.at[t],
                                   sems[t])
                  for t in range(depth)]
            sh = []
            for t in range(depth):
                gh[t].wait()
                sh.append(pltpu.async_copy(rows_v.at[t],
                                           acc.at[dst_v.at[j + t]],
                                           sems[depth + t], add=True))
            for h in sh:
                h.wait()
            return carry

        lax.fori_loop(0, TCH // depth, body, 0)
        for r in range(rem):
            j = TCH - rem + r
            pltpu.async_copy(table.at[src_v.at[j]], rows_v.at[0],
                             sems[0]).wait()
            pltpu.sync_copy(rows_v.at[0], acc.at[dst_v.at[j]], add=True)
        plsc.subcore_barrier()
        pltpu.sync_copy(acc.at[pl.ds(s * RPT, RPT)],
                        out.at[c, pl.ds(s * RPT, RPT)])

    return prop


_R = 512
_G = NP // _R


def _ln(r):
    mu = jnp.mean(r, axis=1, keepdims=True)
    var = jnp.mean((r - mu) ** 2, axis=1, keepdims=True)
    return (r - mu) * lax.rsqrt(var + 1e-5)


def _row_spec(width):
    return pl.BlockSpec((_R, width), lambda i: (i, 0))


def _full_spec(shape):
    return pl.BlockSpec(shape, lambda i: tuple(0 for _ in shape))


def _tc_scale(x, scale_b):
    def body(x_ref, s_ref, o_ref):
        o_ref[...] = x_ref[...] * s_ref[...]

    return pl.pallas_call(
        body,
        grid=(_G,),
        in_specs=[_row_spec(128), _row_spec(128)],
        out_specs=_row_spec(128),
        out_shape=jax.ShapeDtypeStruct((NP, 128), _f32),
    )(x, scale_b)


def _tc_layer1(pa0, pa1, dnin_b, dnout_b, W1, b1):
    def body(a0, a1, di, do, w_ref, b_ref, oa, ob):
        agg = (a0[...] + a1[...]) * di[...]
        t = jnp.dot(agg, w_ref[...], preferred_element_type=_f32)
        t = t + b_ref[0:1, :]
        h = _ln(jnp.maximum(t, 0.0))
        dob = do[...]
        oa[...] = h[:, :128] * dob
        ob[...] = h[:, 128:] * dob

    return pl.pallas_call(
        body,
        grid=(_G,),
        in_specs=[_row_spec(128), _row_spec(128), _row_spec(128),
                  _row_spec(128), _full_spec((128, 256)), _full_spec((8, 256))],
        out_specs=[_row_spec(128), _row_spec(128)],
        out_shape=[jax.ShapeDtypeStruct((NP, 128), _f32),
                   jax.ShapeDtypeStruct((NP, 128), _f32)],
    )(pa0, pa1, dnin_b, dnout_b, W1, b1)


def _tc_layer2(pa0, pa1, pb0, pb1, dnin_b, dnout_b, W2, b2, W3):
    def body(a0, a1, b0, b1_, di, do, w2_ref, bias_ref, w3_ref, og):
        dib = di[...]
        agg_lo = (a0[...] + a1[...]) * dib
        agg_hi = (b0[...] + b1_[...]) * dib
        t = (jnp.dot(agg_lo, w2_ref[:128, :], preferred_element_type=_f32)
             + jnp.dot(agg_hi, w2_ref[128:, :], preferred_element_type=_f32))
        t = t + bias_ref[0:1, :]
        h = _ln(jnp.maximum(t, 0.0))
        dob = do[...]
        hs_lo = h[:, :128] * dob
        hs_hi = h[:, 128:] * dob
        og[...] = (jnp.dot(hs_lo, w3_ref[:128, :], preferred_element_type=_f32)
                   + jnp.dot(hs_hi, w3_ref[128:, :],
                             preferred_element_type=_f32))

    return pl.pallas_call(
        body,
        grid=(_G,),
        in_specs=[_row_spec(128), _row_spec(128), _row_spec(128),
                  _row_spec(128), _row_spec(128), _row_spec(128),
                  _full_spec((256, 256)), _full_spec((8, 256)),
                  _full_spec((256, 64))],
        out_specs=_row_spec(64),
        out_shape=jax.ShapeDtypeStruct((NP, 64), _f32),
    )(pa0, pa1, pb0, pb1, dnin_b, dnout_b, W2, b2, W3)


def _tc_layer3(q0, q1, dnin_b64, b3):
    def body(a0, a1, di, bias_ref, oy):
        agg = (a0[...] + a1[...]) * di[...]
        t = agg + bias_ref[0:1, :]
        oy[...] = _ln(jnp.maximum(t, 0.0))

    return pl.pallas_call(
        body,
        grid=(_G,),
        in_specs=[_row_spec(64), _row_spec(64), _row_spec(64),
                  _full_spec((8, 64))],
        out_specs=_row_spec(64),
        out_shape=jax.ShapeDtypeStruct((NP, 64), _f32),
    )(q0, q1, dnin_b64, b3)


def kernel(features, edge_index, W1, b1, W2, b2, W3, b3):
    src = edge_index[0].astype(_i32).reshape(NW, TCH, CH)
    dst = edge_index[1].astype(_i32).reshape(NW, TCH, CH)

    feats_p = jnp.pad(features, ((0, NP - N), (0, 0)))
    ones_h = jnp.ones((CH, DW), _f32)
    zeros_dw = jnp.zeros((RPT, DW), _f32)
    zeros_128 = jnp.zeros((128, 128), _f32)
    zeros_64 = jnp.zeros((128, 64), _f32)

    cs, cd = _make_count()(src, dst, ones_h, zeros_dw)
    deg_out = jnp.clip(cs[0, :, 0] + cs[1, :, 0], 1.0, None)
    deg_in = jnp.clip(cd[0, :, 0] + cd[1, :, 0], 1.0, None)
    dn_out = deg_out ** -0.5
    dn_in = deg_in ** -0.5
    dn_out_b = jnp.broadcast_to(dn_out[:, None], (NP, 128))
    dn_in_b = jnp.broadcast_to(dn_in[:, None], (NP, 128))
    dn_in_b64 = jnp.broadcast_to(dn_in[:, None], (NP, 64))

    b1_p = jnp.broadcast_to(b1[None, :], (8, 256))
    b2_p = jnp.broadcast_to(b2[None, :], (8, 256))
    b3_p = jnp.broadcast_to(b3[None, :], (8, 64))

    prop128 = _make_propagate(128, 2)
    prop64 = _make_propagate(64, 5)

    h0s = _tc_scale(feats_p, dn_out_b)
    p1 = prop128(h0s, src, dst, zeros_128)
    h1a, h1b = _tc_layer1(p1[0], p1[1], dn_in_b, dn_out_b, W1, b1_p)

    pa = prop128(h1a, src, dst, zeros_128)
    pb = prop128(h1b, src, dst, zeros_128)
    g = _tc_layer2(pa[0], pa[1], pb[0], pb[1], dn_in_b, dn_out_b, W2, b2_p,
                   W3)

    q = prop64(g, src, dst, zeros_64)
    y = _tc_layer3(q[0], q[1], dn_in_b64, b3_p)
    return y[:N]


# speedup vs baseline: 8.4417x; 1.0607x over previous
import functools

import jax
import jax.numpy as jnp
from jax import lax
from jax.experimental import pallas as pl
from jax.experimental.pallas import tpu as pltpu
from jax.experimental.pallas import tpu_sc as plsc

N = 10000
E = 320000
NP = 10240
NC = 2
NS = 16
NW = NC * NS
EPT = E // NW
CH = 80
TCH = EPT // CH
RPT = NP // NS

_f32 = jnp.float32
_i32 = jnp.int32


def _sc_mesh():
    return plsc.VectorSubcoreMesh(core_axis_name="c", subcore_axis_name="s")


DW = 16


def _make_count():
    @functools.partial(
        pl.kernel,
        out_type=[jax.ShapeDtypeStruct((NC, NP, DW), _f32),
                  jax.ShapeDtypeStruct((NC, NP, DW), _f32)],
        mesh=_sc_mesh(),
        scratch_types=[
            pltpu.VMEM((TCH, CH), _i32),
            pltpu.VMEM((TCH, CH), _i32),
            pltpu.VMEM((CH, DW), _f32),
            pltpu.VMEM((RPT, DW), _f32),
            pltpu.VMEM_SHARED((NP, DW), _f32),
            pltpu.VMEM_SHARED((NP, DW), _f32),
        ],
        compiler_params=pltpu.CompilerParams(use_tc_tiling_on_sc=False),
    )
    def cnt(srcs, dsts, ones_h, zeros_h, out_s, out_d,
            src_v, dst_v, ones_v, zv, acc_s, acc_d):
        c = lax.axis_index("c")
        s = lax.axis_index("s")
        w = c * NS + s
        pltpu.sync_copy(srcs.at[w], src_v)
        pltpu.sync_copy(dsts.at[w], dst_v)
        pltpu.sync_copy(ones_h, ones_v)
        pltpu.sync_copy(zeros_h, zv)
        pltpu.sync_copy(zv, acc_s.at[pl.ds(s * RPT, RPT)])
        pltpu.sync_copy(zv, acc_d.at[pl.ds(s * RPT, RPT)])
        plsc.subcore_barrier()

        def body(j, carry):
            pltpu.sync_copy(ones_v, acc_s.at[src_v.at[j]], add=True)
            pltpu.sync_copy(ones_v, acc_d.at[dst_v.at[j]], add=True)
            return carry

        lax.fori_loop(0, TCH, body, 0)
        plsc.subcore_barrier()
        pltpu.sync_copy(acc_s.at[pl.ds(s * RPT, RPT)],
                        out_s.at[c, pl.ds(s * RPT, RPT)])
        pltpu.sync_copy(acc_d.at[pl.ds(s * RPT, RPT)],
                        out_d.at[c, pl.ds(s * RPT, RPT)])

    return cnt


def _make_propagate(width, depth):
    rem = TCH % depth

    @functools.partial(
        pl.kernel,
        out_type=jax.ShapeDtypeStruct((NC, NP, width), _f32),
        mesh=_sc_mesh(),
        scratch_types=[
            pltpu.VMEM((depth, CH, width), _f32),
            pltpu.VMEM((TCH, CH), _i32),
            pltpu.VMEM((TCH, CH), _i32),
            pltpu.VMEM_SHARED((NP, width), _f32),
        ] + [pltpu.SemaphoreType.DMA] * (2 * depth),
        compiler_params=pltpu.CompilerParams(use_tc_tiling_on_sc=False),
    )
    def prop(table, srcs, dsts, zrows, out, rows_v, src_v, dst_v, acc,
             *sems):
        c = lax.axis_index("c")
        s = lax.axis_index("s")
        w = c * NS + s
        pltpu.sync_copy(srcs.at[w], src_v)
        pltpu.sync_copy(dsts.at[w], dst_v)
        pltpu.sync_copy(zrows.at[pl.ds(0, CH)], rows_v.at[0])
        for t in range(RPT // CH):
            pltpu.sync_copy(rows_v.at[0],
                            acc.at[pl.ds(s * RPT + t * CH, CH)])
        plsc.subcore_barrier()

        def body(i, carry):
            j = i * depth
            gh = [pltpu.async_copy(table.at[src_v.at[j + t]], rows_v.at[t],
                                   sems[t])
                  for t in range(depth)]
            sh = []
            for t in range(depth):
                gh[t].wait()
                sh.append(pltpu.async_copy(rows_v.at[t],
                                           acc.at[dst_v.at[j + t]],
                                           sems[depth + t], add=True))
            for h in sh:
                h.wait()
            return carry

        lax.fori_loop(0, TCH // depth, body, 0)
        for r in range(rem):
            j = TCH - rem + r
            pltpu.async_copy(table.at[src_v.at[j]], rows_v.at[0],
                             sems[0]).wait()
            pltpu.sync_copy(rows_v.at[0], acc.at[dst_v.at[j]], add=True)
        plsc.subcore_barrier()
        pltpu.sync_copy(acc.at[pl.ds(s * RPT, RPT)],
                        out.at[c, pl.ds(s * RPT, RPT)])

    return prop


def _make_propagate2(width, depth):
    rem = TCH % depth

    @functools.partial(
        pl.kernel,
        out_type=[jax.ShapeDtypeStruct((NC, NP, width), _f32),
                  jax.ShapeDtypeStruct((NC, NP, width), _f32)],
        mesh=_sc_mesh(),
        scratch_types=[
            pltpu.VMEM((depth, CH, width), _f32),
            pltpu.VMEM((TCH, CH), _i32),
            pltpu.VMEM((TCH, CH), _i32),
            pltpu.VMEM_SHARED((NP, width), _f32),
        ] + [pltpu.SemaphoreType.DMA] * (2 * depth),
        compiler_params=pltpu.CompilerParams(use_tc_tiling_on_sc=False),
    )
    def prop2(ta, tb, srcs, dsts, zrows, out_a, out_b,
              rows_v, src_v, dst_v, acc, *sems):
        c = lax.axis_index("c")
        s = lax.axis_index("s")
        w = c * NS + s
        pltpu.sync_copy(srcs.at[w], src_v)
        pltpu.sync_copy(dsts.at[w], dst_v)

        def zero_acc():
            pltpu.sync_copy(zrows.at[pl.ds(0, CH)], rows_v.at[0])
            for t in range(RPT // CH):
                pltpu.sync_copy(rows_v.at[0],
                                acc.at[pl.ds(s * RPT + t * CH, CH)])

        def one_pass(table, out):
            def body(i, carry):
                j = i * depth
                gh = [pltpu.async_copy(table.at[src_v.at[j + t]],
                                       rows_v.at[t], sems[t])
                      for t in range(depth)]
                sh = []
                for t in range(depth):
                    gh[t].wait()
                    sh.append(pltpu.async_copy(rows_v.at[t],
                                               acc.at[dst_v.at[j + t]],
                                               sems[depth + t], add=True))
                for h in sh:
                    h.wait()
                return carry

            lax.fori_loop(0, TCH // depth, body, 0)
            for r in range(rem):
                j = TCH - rem + r
                pltpu.async_copy(table.at[src_v.at[j]], rows_v.at[0],
                                 sems[0]).wait()
                pltpu.sync_copy(rows_v.at[0], acc.at[dst_v.at[j]], add=True)
            plsc.subcore_barrier()
            pltpu.sync_copy(acc.at[pl.ds(s * RPT, RPT)],
                            out.at[c, pl.ds(s * RPT, RPT)])

        zero_acc()
        plsc.subcore_barrier()
        one_pass(ta, out_a)
        plsc.subcore_barrier()
        zero_acc()
        plsc.subcore_barrier()
        one_pass(tb, out_b)

    return prop2


_R = 512
_G = NP // _R


def _ln(r):
    mu = jnp.mean(r, axis=1, keepdims=True)
    var = jnp.mean((r - mu) ** 2, axis=1, keepdims=True)
    return (r - mu) * lax.rsqrt(var + 1e-5)


def _row_spec(width):
    return pl.BlockSpec((_R, width), lambda i: (i, 0))


def _pair_spec(width):
    return pl.BlockSpec((NC, _R, width), lambda i: (0, i, 0))


def _full_spec(shape):
    return pl.BlockSpec(shape, lambda i: tuple(0 for _ in shape))


def _dn(cnt_ref):
    deg = jnp.maximum(cnt_ref[0] + cnt_ref[1], 1.0)
    return lax.rsqrt(deg[:, 0:1])


def _tc_scale(x, cs):
    def body(x_ref, cs_ref, o_ref):
        o_ref[...] = x_ref[...] * _dn(cs_ref)

    return pl.pallas_call(
        body,
        grid=(_G,),
        in_specs=[_row_spec(128), _pair_spec(DW)],
        out_specs=_row_spec(128),
        out_shape=jax.ShapeDtypeStruct((NP, 128), _f32),
    )(x, cs)


def _tc_layer1(p1, cs, cd, W1, b1):
    def body(p_ref, cs_ref, cd_ref, w_ref, b_ref, oa, ob):
        agg = (p_ref[0] + p_ref[1]) * _dn(cd_ref)
        t = jnp.dot(agg, w_ref[...], preferred_element_type=_f32)
        t = t + b_ref[0:1, :]
        h = _ln(jnp.maximum(t, 0.0))
        dob = _dn(cs_ref)
        oa[...] = h[:, :128] * dob
        ob[...] = h[:, 128:] * dob

    return pl.pallas_call(
        body,
        grid=(_G,),
        in_specs=[_pair_spec(128), _pair_spec(DW), _pair_spec(DW),
                  _full_spec((128, 256)), _full_spec((8, 256))],
        out_specs=[_row_spec(128), _row_spec(128)],
        out_shape=[jax.ShapeDtypeStruct((NP, 128), _f32),
                   jax.ShapeDtypeStruct((NP, 128), _f32)],
    )(p1, cs, cd, W1, b1)


def _tc_layer2(pa, pb, cs, cd, W2, b2, W3):
    def body(pa_ref, pb_ref, cs_ref, cd_ref, w2_ref, bias_ref, w3_ref, og):
        dib = _dn(cd_ref)
        agg_lo = (pa_ref[0] + pa_ref[1]) * dib
        agg_hi = (pb_ref[0] + pb_ref[1]) * dib
        t = (jnp.dot(agg_lo, w2_ref[:128, :], preferred_element_type=_f32)
             + jnp.dot(agg_hi, w2_ref[128:, :], preferred_element_type=_f32))
        t = t + bias_ref[0:1, :]
        h = _ln(jnp.maximum(t, 0.0))
        dob = _dn(cs_ref)
        hs_lo = h[:, :128] * dob
        hs_hi = h[:, 128:] * dob
        og[...] = (jnp.dot(hs_lo, w3_ref[:128, :], preferred_element_type=_f32)
                   + jnp.dot(hs_hi, w3_ref[128:, :],
                             preferred_element_type=_f32))

    return pl.pallas_call(
        body,
        grid=(_G,),
        in_specs=[_pair_spec(128), _pair_spec(128), _pair_spec(DW),
                  _pair_spec(DW), _full_spec((256, 256)),
                  _full_spec((8, 256)), _full_spec((256, 64))],
        out_specs=_row_spec(64),
        out_shape=jax.ShapeDtypeStruct((NP, 64), _f32),
    )(pa, pb, cs, cd, W2, b2, W3)


def _tc_layer3(q, cd, b3):
    def body(q_ref, cd_ref, bias_ref, oy):
        agg = (q_ref[0] + q_ref[1]) * _dn(cd_ref)
        t = agg + bias_ref[0:1, :]
        oy[...] = _ln(jnp.maximum(t, 0.0))

    return pl.pallas_call(
        body,
        grid=(_G,),
        in_specs=[_pair_spec(64), _pair_spec(DW), _full_spec((8, 64))],
        out_specs=_row_spec(64),
        out_shape=jax.ShapeDtypeStruct((NP, 64), _f32),
    )(q, cd, b3)


def kernel(features, edge_index, W1, b1, W2, b2, W3, b3):
    src = edge_index[0].astype(_i32).reshape(NW, TCH, CH)
    dst = edge_index[1].astype(_i32).reshape(NW, TCH, CH)

    feats_p = jnp.pad(features, ((0, NP - N), (0, 0)))
    ones_h = jnp.ones((CH, DW), _f32)
    zeros_dw = jnp.zeros((RPT, DW), _f32)
    zeros_128 = jnp.zeros((128, 128), _f32)
    zeros_64 = jnp.zeros((128, 64), _f32)

    b1_p = jnp.broadcast_to(b1[None, :], (8, 256))
    b2_p = jnp.broadcast_to(b2[None, :], (8, 256))
    b3_p = jnp.broadcast_to(b3[None, :], (8, 64))

    prop128 = _make_propagate(128, 2)
    prop2x128 = _make_propagate2(128, 2)
    prop64 = _make_propagate(64, 5)

    cs, cd = _make_count()(src, dst, ones_h, zeros_dw)

    h0s = _tc_scale(feats_p, cs)
    p1 = prop128(h0s, src, dst, zeros_128)
    h1a, h1b = _tc_layer1(p1, cs, cd, W1, b1_p)

    pa, pb = prop2x128(h1a, h1b, src, dst, zeros_128)
    g = _tc_layer2(pa, pb, cs, cd, W2, b2_p, W3)

    q = prop64(g, src, dst, zeros_64)
    y = _tc_layer3(q, cd, b3_p)
    return y[:N]


# speedup vs baseline: 8.4534x; 1.0014x over previous
import functools

import jax
import jax.numpy as jnp
from jax import lax
from jax.experimental import pallas as pl
from jax.experimental.pallas import tpu as pltpu
from jax.experimental.pallas import tpu_sc as plsc

N = 10000
E = 320000
NP = 10240
NC = 2
NS = 16
NW = NC * NS
EPT = E // NW
CH = 80
TCH = EPT // CH
RPT = NP // NS

_f32 = jnp.float32
_i32 = jnp.int32


def _sc_mesh():
    return plsc.VectorSubcoreMesh(core_axis_name="c", subcore_axis_name="s")


DW = 16


def _make_count():
    @functools.partial(
        pl.kernel,
        out_type=[jax.ShapeDtypeStruct((NC, NP, DW), _f32),
                  jax.ShapeDtypeStruct((NC, NP, DW), _f32)],
        mesh=_sc_mesh(),
        scratch_types=[
            pltpu.VMEM((TCH, CH), _i32),
            pltpu.VMEM((TCH, CH), _i32),
            pltpu.VMEM((CH, DW), _f32),
            pltpu.VMEM((RPT, DW), _f32),
            pltpu.VMEM_SHARED((NP, DW), _f32),
            pltpu.VMEM_SHARED((NP, DW), _f32),
        ],
        compiler_params=pltpu.CompilerParams(use_tc_tiling_on_sc=False),
    )
    def cnt(srcs, dsts, ones_h, zeros_h, out_s, out_d,
            src_v, dst_v, ones_v, zv, acc_s, acc_d):
        c = lax.axis_index("c")
        s = lax.axis_index("s")
        w = c * NS + s
        pltpu.sync_copy(srcs.at[w], src_v)
        pltpu.sync_copy(dsts.at[w], dst_v)
        pltpu.sync_copy(ones_h, ones_v)
        pltpu.sync_copy(zeros_h, zv)
        pltpu.sync_copy(zv, acc_s.at[pl.ds(s * RPT, RPT)])
        pltpu.sync_copy(zv, acc_d.at[pl.ds(s * RPT, RPT)])
        plsc.subcore_barrier()

        def body(j, carry):
            pltpu.sync_copy(ones_v, acc_s.at[src_v.at[j]], add=True)
            pltpu.sync_copy(ones_v, acc_d.at[dst_v.at[j]], add=True)
            return carry

        lax.fori_loop(0, TCH, body, 0)
        plsc.subcore_barrier()
        pltpu.sync_copy(acc_s.at[pl.ds(s * RPT, RPT)],
                        out_s.at[c, pl.ds(s * RPT, RPT)])
        pltpu.sync_copy(acc_d.at[pl.ds(s * RPT, RPT)],
                        out_d.at[c, pl.ds(s * RPT, RPT)])

    return cnt


def _make_propagate(width, depth):
    rem = TCH % depth

    @functools.partial(
        pl.kernel,
        out_type=jax.ShapeDtypeStruct((NC, NP, width), _f32),
        mesh=_sc_mesh(),
        scratch_types=[
            pltpu.VMEM((depth, CH, width), _f32),
            pltpu.VMEM((TCH, CH), _i32),
            pltpu.VMEM((TCH, CH), _i32),
            pltpu.VMEM_SHARED((NP, width), _f32),
        ] + [pltpu.SemaphoreType.DMA] * (2 * depth),
        compiler_params=pltpu.CompilerParams(use_tc_tiling_on_sc=False),
    )
    def prop(table, srcs, dsts, zrows, out, rows_v, src_v, dst_v, acc,
             *sems):
        c = lax.axis_index("c")
        s = lax.axis_index("s")
        w = c * NS + s
        pltpu.sync_copy(srcs.at[w], src_v)
        pltpu.sync_copy(dsts.at[w], dst_v)
        pltpu.sync_copy(zrows.at[pl.ds(0, CH)], rows_v.at[0])
        for t in range(RPT // CH):
            pltpu.sync_copy(rows_v.at[0],
                            acc.at[pl.ds(s * RPT + t * CH, CH)])
        plsc.subcore_barrier()

        def body(i, carry):
            j = i * depth
            for t in range(depth):
                @pl.when(i > 0)
                def _drain(t=t):
                    pltpu.make_async_copy(
                        rows_v.at[t], acc.at[dst_v.at[j - depth + t]],
                        sems[depth + t]).wait()
            gh = [pltpu.async_copy(table.at[src_v.at[j + t]], rows_v.at[t],
                                   sems[t])
                  for t in range(depth)]
            for t in range(depth):
                gh[t].wait()
                pltpu.async_copy(rows_v.at[t], acc.at[dst_v.at[j + t]],
                                 sems[depth + t], add=True)
            return carry

        nit = TCH // depth
        lax.fori_loop(0, nit, body, 0)
        for t in range(depth):
            pltpu.make_async_copy(
                rows_v.at[t], acc.at[dst_v.at[(nit - 1) * depth + t]],
                sems[depth + t]).wait()
        for r in range(rem):
            j = TCH - rem + r
            pltpu.async_copy(table.at[src_v.at[j]], rows_v.at[0],
                             sems[0]).wait()
            pltpu.sync_copy(rows_v.at[0], acc.at[dst_v.at[j]], add=True)
        plsc.subcore_barrier()
        pltpu.sync_copy(acc.at[pl.ds(s * RPT, RPT)],
                        out.at[c, pl.ds(s * RPT, RPT)])

    return prop


def _make_propagate2(width, depth):
    rem = TCH % depth

    @functools.partial(
        pl.kernel,
        out_type=[jax.ShapeDtypeStruct((NC, NP, width), _f32),
                  jax.ShapeDtypeStruct((NC, NP, width), _f32)],
        mesh=_sc_mesh(),
        scratch_types=[
            pltpu.VMEM((depth, CH, width), _f32),
            pltpu.VMEM((TCH, CH), _i32),
            pltpu.VMEM((TCH, CH), _i32),
            pltpu.VMEM_SHARED((NP, width), _f32),
        ] + [pltpu.SemaphoreType.DMA] * (2 * depth),
        compiler_params=pltpu.CompilerParams(use_tc_tiling_on_sc=False),
    )
    def prop2(ta, tb, srcs, dsts, zrows, out_a, out_b,
              rows_v, src_v, dst_v, acc, *sems):
        c = lax.axis_index("c")
        s = lax.axis_index("s")
        w = c * NS + s
        pltpu.sync_copy(srcs.at[w], src_v)
        pltpu.sync_copy(dsts.at[w], dst_v)

        def zero_acc():
            pltpu.sync_copy(zrows.at[pl.ds(0, CH)], rows_v.at[0])
            for t in range(RPT // CH):
                pltpu.sync_copy(rows_v.at[0],
                                acc.at[pl.ds(s * RPT + t * CH, CH)])

        def one_pass(table, out):
            def body(i, carry):
                j = i * depth
                for t in range(depth):
                    @pl.when(i > 0)
                    def _drain(t=t):
                        pltpu.make_async_copy(
                            rows_v.at[t], acc.at[dst_v.at[j - depth + t]],
                            sems[depth + t]).wait()
                gh = [pltpu.async_copy(table.at[src_v.at[j + t]],
                                       rows_v.at[t], sems[t])
                      for t in range(depth)]
                for t in range(depth):
                    gh[t].wait()
                    pltpu.async_copy(rows_v.at[t], acc.at[dst_v.at[j + t]],
                                     sems[depth + t], add=True)
                return carry

            nit = TCH // depth
            lax.fori_loop(0, nit, body, 0)
            for t in range(depth):
                pltpu.make_async_copy(
                    rows_v.at[t], acc.at[dst_v.at[(nit - 1) * depth + t]],
                    sems[depth + t]).wait()
            for r in range(rem):
                j = TCH - rem + r
                pltpu.async_copy(table.at[src_v.at[j]], rows_v.at[0],
                                 sems[0]).wait()
                pltpu.sync_copy(rows_v.at[0], acc.at[dst_v.at[j]], add=True)
            plsc.subcore_barrier()
            pltpu.sync_copy(acc.at[pl.ds(s * RPT, RPT)],
                            out.at[c, pl.ds(s * RPT, RPT)])

        zero_acc()
        plsc.subcore_barrier()
        one_pass(ta, out_a)
        plsc.subcore_barrier()
        zero_acc()
        plsc.subcore_barrier()
        one_pass(tb, out_b)

    return prop2


_R = 512
_G = NP // _R


def _ln(r):
    mu = jnp.mean(r, axis=1, keepdims=True)
    var = jnp.mean((r - mu) ** 2, axis=1, keepdims=True)
    return (r - mu) * lax.rsqrt(var + 1e-5)


def _row_spec(width):
    return pl.BlockSpec((_R, width), lambda i: (i, 0))


def _pair_spec(width):
    return pl.BlockSpec((NC, _R, width), lambda i: (0, i, 0))


def _full_spec(shape):
    return pl.BlockSpec(shape, lambda i: tuple(0 for _ in shape))


def _dn(cnt_ref):
    deg = jnp.maximum(cnt_ref[0] + cnt_ref[1], 1.0)
    return lax.rsqrt(deg[:, 0:1])


def _tc_scale(x, cs):
    def body(x_ref, cs_ref, o_ref):
        o_ref[...] = x_ref[...] * _dn(cs_ref)

    return pl.pallas_call(
        body,
        grid=(_G,),
        in_specs=[_row_spec(128), _pair_spec(DW)],
        out_specs=_row_spec(128),
        out_shape=jax.ShapeDtypeStruct((NP, 128), _f32),
    )(x, cs)


def _tc_layer1(p1, cs, cd, W1, b1):
    def body(p_ref, cs_ref, cd_ref, w_ref, b_ref, oa, ob):
        agg = (p_ref[0] + p_ref[1]) * _dn(cd_ref)
        t = jnp.dot(agg, w_ref[...], preferred_element_type=_f32)
        t = t + b_ref[0:1, :]
        h = _ln(jnp.maximum(t, 0.0))
        dob = _dn(cs_ref)
        oa[...] = h[:, :128] * dob
        ob[...] = h[:, 128:] * dob

    return pl.pallas_call(
        body,
        grid=(_G,),
        in_specs=[_pair_spec(128), _pair_spec(DW), _pair_spec(DW),
                  _full_spec((128, 256)), _full_spec((8, 256))],
        out_specs=[_row_spec(128), _row_spec(128)],
        out_shape=[jax.ShapeDtypeStruct((NP, 128), _f32),
                   jax.ShapeDtypeStruct((NP, 128), _f32)],
    )(p1, cs, cd, W1, b1)


def _tc_layer2(pa, pb, cs, cd, W2, b2, W3):
    def body(pa_ref, pb_ref, cs_ref, cd_ref, w2_ref, bias_ref, w3_ref, og):
        dib = _dn(cd_ref)
        agg_lo = (pa_ref[0] + pa_ref[1]) * dib
        agg_hi = (pb_ref[0] + pb_ref[1]) * dib
        t = (jnp.dot(agg_lo, w2_ref[:128, :], preferred_element_type=_f32)
             + jnp.dot(agg_hi, w2_ref[128:, :], preferred_element_type=_f32))
        t = t + bias_ref[0:1, :]
        h = _ln(jnp.maximum(t, 0.0))
        dob = _dn(cs_ref)
        hs_lo = h[:, :128] * dob
        hs_hi = h[:, 128:] * dob
        og[...] = (jnp.dot(hs_lo, w3_ref[:128, :], preferred_element_type=_f32)
                   + jnp.dot(hs_hi, w3_ref[128:, :],
                             preferred_element_type=_f32))

    return pl.pallas_call(
        body,
        grid=(_G,),
        in_specs=[_pair_spec(128), _pair_spec(128), _pair_spec(DW),
                  _pair_spec(DW), _full_spec((256, 256)),
                  _full_spec((8, 256)), _full_spec((256, 64))],
        out_specs=_row_spec(64),
        out_shape=jax.ShapeDtypeStruct((NP, 64), _f32),
    )(pa, pb, cs, cd, W2, b2, W3)


def _tc_layer3(q, cd, b3):
    def body(q_ref, cd_ref, bias_ref, oy):
        agg = (q_ref[0] + q_ref[1]) * _dn(cd_ref)
        t = agg + bias_ref[0:1, :]
        oy[...] = _ln(jnp.maximum(t, 0.0))

    return pl.pallas_call(
        body,
        grid=(_G,),
        in_specs=[_pair_spec(64), _pair_spec(DW), _full_spec((8, 64))],
        out_specs=_row_spec(64),
        out_shape=jax.ShapeDtypeStruct((NP, 64), _f32),
    )(q, cd, b3)


def kernel(features, edge_index, W1, b1, W2, b2, W3, b3):
    src = edge_index[0].astype(_i32).reshape(NW, TCH, CH)
    dst = edge_index[1].astype(_i32).reshape(NW, TCH, CH)

    feats_p = jnp.pad(features, ((0, NP - N), (0, 0)))
    ones_h = jnp.ones((CH, DW), _f32)
    zeros_dw = jnp.zeros((RPT, DW), _f32)
    zeros_128 = jnp.zeros((128, 128), _f32)
    zeros_64 = jnp.zeros((128, 64), _f32)

    b1_p = jnp.broadcast_to(b1[None, :], (8, 256))
    b2_p = jnp.broadcast_to(b2[None, :], (8, 256))
    b3_p = jnp.broadcast_to(b3[None, :], (8, 64))

    prop128 = _make_propagate(128, 2)
    prop2x128 = _make_propagate2(128, 2)
    prop64 = _make_propagate(64, 5)

    cs, cd = _make_count()(src, dst, ones_h, zeros_dw)

    h0s = _tc_scale(feats_p, cs)
    p1 = prop128(h0s, src, dst, zeros_128)
    h1a, h1b = _tc_layer1(p1, cs, cd, W1, b1_p)

    pa, pb = prop2x128(h1a, h1b, src, dst, zeros_128)
    g = _tc_layer2(pa, pb, cs, cd, W2, b2_p, W3)

    q = prop64(g, src, dst, zeros_64)
    y = _tc_layer3(q, cd, b3_p)
    return y[:N]
